# baseline (speedup 1.0000x reference)
import functools

import jax
import jax.numpy as jnp
from jax import lax
from jax.experimental import pallas as pl
from jax.experimental.pallas import tpu as pltpu
from jax.experimental.pallas import tpu_sc as plsc

E = 128
D = 128
NC = 2
NS = 16
L = 16
NW = NC * NS


def _project_tables_body(t_ref, w_ref, b_ref, out_ref):
    tp = lax.dot_general(
        t_ref[...], w_ref[...], (((1,), (1,)), ((), ())),
        preferred_element_type=jnp.float32)
    rows = lax.broadcasted_iota(jnp.int32, (4 * E, 1), 0)
    in_w_seg = (rows >= 2 * E) & (rows < 3 * E)
    out_ref[...] = tp + jnp.where(in_w_seg, b_ref[...], jnp.float32(0.0))


def _project_tables(tables, w, b):
    return pl.pallas_call(
        _project_tables_body,
        out_shape=jax.ShapeDtypeStruct((4 * E, D), jnp.float32),
    )(tables, w, b.reshape(1, D))


def _pick_chunk(rows_per_worker):
    c = 0
    for m in range(1, rows_per_worker // L + 1):
        cand = L * m
        if rows_per_worker % cand == 0 and cand <= 256:
            c = cand
    return c


def _sc_lookup_body(n_rows, n_per_batch, chunk, n_scales, boxes_hbm, tp_hbm,
                    scales_hbm, out_hbm, tp_v, box_v, out_v, sc_v):
    rpw = n_rows // NW
    wid = lax.axis_index("s") * NC + lax.axis_index("c")
    base = wid * rpw
    batch = base // n_per_batch

    pltpu.sync_copy(tp_hbm, tp_v)
    pltpu.sync_copy(scales_hbm, sc_v.at[pl.ds(0, n_scales)])
    iota = lax.broadcasted_iota(jnp.int32, (L,), 0)
    h_img = plsc.load_gather(sc_v, [jnp.full((L,), 2 * batch, jnp.int32)])
    w_img = plsc.load_gather(sc_v, [jnp.full((L,), 2 * batch + 1, jnp.int32)])
    ef = jnp.float32(E)
    emax = jnp.float32(E - 1)

    @pl.loop(0, rpw // chunk)
    def _chunk(g):
        row0 = base + g * chunk
        pltpu.sync_copy(boxes_hbm.at[pl.ds(row0 * 8, chunk * 8)], box_v)
        for j in range(chunk // L):
            rows_local = j * L + iota
            rows_k = rows_local * 8

            def col(k):
                return plsc.load_gather(box_v, [rows_k + k])

            x0, x1, x2, x3 = col(0), col(2), col(4), col(6)
            y0, y1, y2, y3 = col(1), col(3), col(5), col(7)
            xminf = jnp.minimum(jnp.minimum(x0, x1), jnp.minimum(x2, x3))
            xmaxf = jnp.maximum(jnp.maximum(x0, x1), jnp.maximum(x2, x3))
            yminf = jnp.minimum(jnp.minimum(y0, y1), jnp.minimum(y2, y3))
            ymaxf = jnp.maximum(jnp.maximum(y0, y1), jnp.maximum(y2, y3))

            def to_idx(v, denom):
                scaled = (v / denom) * ef
                return jnp.clip(scaled, jnp.float32(0.0), emax).astype(jnp.int32)

            ixmin = to_idx(xminf, w_img)
            ixmax = to_idx(xmaxf, w_img)
            iymin = to_idx(yminf, h_img)
            iymax = to_idx(ymaxf, h_img)
            a0 = ixmin * D
            a1 = (iymin + E) * D
            a2 = ixmax * D
            a3 = (iymax + E) * D
            a4 = ((ixmax - ixmin) + 2 * E) * D
            a5 = ((iymax - iymin) + 3 * E) * D
            rows_d = rows_local * D

            @pl.loop(0, D, unroll=4)
            def _col(cc):
                ccs = jnp.full((L,), cc, jnp.int32)
                acc = plsc.load_gather(tp_v, [a0 + ccs])
                acc = acc + plsc.load_gather(tp_v, [a1 + ccs])
                acc = acc + plsc.load_gather(tp_v, [a2 + ccs])
                acc = acc + plsc.load_gather(tp_v, [a3 + ccs])
                acc = acc + plsc.load_gather(tp_v, [a4 + ccs])
                acc = acc + plsc.load_gather(tp_v, [a5 + ccs])
                acc = jnp.maximum(acc, jnp.float32(0.0))
                plsc.store_scatter(out_v, [rows_d + ccs], acc)

        pltpu.sync_copy(out_v, out_hbm.at[pl.ds(row0 * D, chunk * D)])


def kernel(boxes, img_shapes, x_emb, y_emb, w_emb, h_emb, W, b):
    B, N, K = boxes.shape
    n_rows = B * N
    tables = jnp.concatenate([x_emb, y_emb, w_emb, h_emb], axis=0)
    tp = _project_tables(tables, W, b)

    rpw = n_rows // NW
    chunk = _pick_chunk(rpw)
    boxes2 = boxes.reshape(n_rows * K)

    mesh = plsc.VectorSubcoreMesh(core_axis_name="c", subcore_axis_name="s")
    body = functools.partial(_sc_lookup_body, n_rows, N, chunk, B * 2)
    out = pl.kernel(
        body,
        out_type=jax.ShapeDtypeStruct((n_rows * D,), jnp.float32),
        mesh=mesh,
        compiler_params=pltpu.CompilerParams(needs_layout_passes=False),
        scratch_types=[
            pltpu.VMEM((4 * E * D,), jnp.float32),
            pltpu.VMEM((chunk * K,), jnp.float32),
            pltpu.VMEM((chunk * D,), jnp.float32),
            pltpu.VMEM((max(B * 2, 128),), jnp.float32),
        ],
    )(boxes2, tp.reshape(4 * E * D), img_shapes.reshape(B * 2))
    return out.reshape(B, N, D)


# speedup vs baseline: 3.8136x; 3.8136x over previous
import functools

import jax
import jax.numpy as jnp
from jax import lax
from jax.experimental import pallas as pl
from jax.experimental.pallas import tpu as pltpu
from jax.experimental.pallas import tpu_sc as plsc

E = 128
D = 128
NC = 2
NS = 16
L = 16
NW = NC * NS
DW = D // 2


def _project_tables_body(t_ref, w_ref, b_ref, out_ref):
    tp = lax.dot_general(
        t_ref[...], w_ref[...], (((1,), (1,)), ((), ())),
        preferred_element_type=jnp.float32)
    rows = lax.broadcasted_iota(jnp.int32, (4 * E, 1), 0)
    in_w_seg = (rows >= 2 * E) & (rows < 3 * E)
    out_ref[...] = tp + jnp.where(in_w_seg, b_ref[...], jnp.float32(0.0))


def _project_tables(tables, w, b):
    return pl.pallas_call(
        _project_tables_body,
        out_shape=jax.ShapeDtypeStruct((4 * E, D), jnp.float32),
    )(tables, w, b.reshape(1, D))


def _pick_chunk(rows_per_worker):
    c = 0
    for m in range(1, rows_per_worker // L + 1):
        cand = L * m
        if rows_per_worker % cand == 0 and cand <= 128:
            c = cand
    return c


def _sc_lookup_body(n_rows, n_per_batch, chunk, n_scales, coords_hbm, tp_hbm,
                    scales_hbm, out_hbm, box_v, idx_v, gath_v, out_v, sc_v,
                    gsem):
    rpw = n_rows // NW
    nchunks = rpw // chunk
    wid = lax.axis_index("s") * NC + lax.axis_index("c")
    base = wid * rpw
    batch = base // n_per_batch

    pltpu.sync_copy(scales_hbm, sc_v.at[pl.ds(0, n_scales)])
    iota = lax.broadcasted_iota(jnp.int32, (L,), 0)
    h_img = plsc.load_gather(sc_v, [jnp.full((L,), 2 * batch, jnp.int32)])
    w_img = plsc.load_gather(sc_v, [jnp.full((L,), 2 * batch + 1, jnp.int32)])
    ef = jnp.float32(E)
    emax = jnp.float32(E - 1)

    @pl.loop(0, nchunks)
    def _chunk(g):
        row0 = base + g * chunk
        pltpu.sync_copy(coords_hbm.at[pl.ds(row0 * 8, chunk * 8)], box_v)

        for j in range(chunk // L):
            rows_k = (j * L + iota) * 8

            def coord(k):
                return plsc.load_gather(box_v, [rows_k + k])

            x0, x1, x2, x3 = coord(0), coord(2), coord(4), coord(6)
            y0, y1, y2, y3 = coord(1), coord(3), coord(5), coord(7)
            xminf = jnp.minimum(jnp.minimum(x0, x1), jnp.minimum(x2, x3))
            xmaxf = jnp.maximum(jnp.maximum(x0, x1), jnp.maximum(x2, x3))
            yminf = jnp.minimum(jnp.minimum(y0, y1), jnp.minimum(y2, y3))
            ymaxf = jnp.maximum(jnp.maximum(y0, y1), jnp.maximum(y2, y3))

            def to_idx(v, denom):
                scaled = (v / denom) * ef
                return jnp.clip(scaled, jnp.float32(0.0), emax).astype(jnp.int32)

            ixmin = to_idx(xminf, w_img)
            ixmax = to_idx(xmaxf, w_img)
            iymin = to_idx(yminf, h_img)
            iymax = to_idx(ymaxf, h_img)
            sl = pl.ds(j * L, L)
            idx_v[0, sl] = ixmin
            idx_v[1, sl] = iymin + E
            idx_v[2, sl] = ixmax
            idx_v[3, sl] = iymax + E
            idx_v[4, sl] = (ixmax - ixmin) + 2 * E
            idx_v[5, sl] = (iymax - iymin) + 3 * E

        for t in range(6):
            pltpu.async_copy(tp_hbm.at[idx_v.at[t]], gath_v.at[t], gsem)
        for t in range(6):
            pltpu.make_async_copy(tp_hbm.at[idx_v.at[t]], gath_v.at[t],
                                  gsem).wait()

        @pl.loop(0, chunk, unroll=2)
        def _acc(r):
            for w in range(D // L):
                s = pl.ds(w * L, L)

                def gb(t):
                    return gath_v[t, r, s]

                acc = ((gb(0) + gb(1)) + (gb(2) + gb(3))) + (gb(4) + gb(5))
                out_v[r, s] = jnp.maximum(acc, jnp.float32(0.0))

        pltpu.sync_copy(out_v, out_hbm.at[pl.ds(row0, chunk)])


def kernel(boxes, img_shapes, x_emb, y_emb, w_emb, h_emb, W, b):
    B, N, K = boxes.shape
    n_rows = B * N
    tables = jnp.concatenate([x_emb, y_emb, w_emb, h_emb], axis=0)
    tp = _project_tables(tables, W, b)

    rpw = n_rows // NW
    chunk = _pick_chunk(rpw)
    boxes2 = boxes.reshape(n_rows * K)

    mesh = plsc.VectorSubcoreMesh(core_axis_name="c", subcore_axis_name="s")
    body = functools.partial(_sc_lookup_body, n_rows, N, chunk, B * 2)
    out_packed = pl.kernel(
        body,
        out_type=jax.ShapeDtypeStruct((n_rows, D), jnp.float32),
        mesh=mesh,
        compiler_params=pltpu.CompilerParams(needs_layout_passes=False),
        scratch_types=[
            pltpu.VMEM((chunk * K,), jnp.float32),
            pltpu.VMEM((6, chunk), jnp.int32),
            pltpu.VMEM((6, chunk, D), jnp.float32),
            pltpu.VMEM((chunk, D), jnp.float32),
            pltpu.VMEM((max(B * 2, 128),), jnp.float32),
            pltpu.SemaphoreType.DMA,
        ],
    )(boxes2, tp, img_shapes.reshape(B * 2))
    return out_packed.reshape(B, N, D)
